# Initial kernel scaffold; baseline (speedup 1.0000x reference)
import jax
import jax.numpy as jnp
from jax.experimental import pallas as pl


def kernel(nodes, edges, globs, edge_idx, params):
    raise NotImplementedError("write your pallas kernel here")


# speedup vs baseline: 3.5993x; 3.5993x over previous
import functools

import jax
import jax.numpy as jnp
from jax import lax
from jax.experimental import pallas as pl
from jax.experimental.pallas import tpu as pltpu
from jax.experimental.pallas import tpu_sc as plsc

_N = 10000
_DEG = 32
_E = _N * _DEG
_DN = 128
_DE = 16
_DG = 16
_H = 64

_NC = 2
_NS = 16
_NW = _NC * _NS
_EPW = _E // _NW
_GCH = 100
_GIT = _EPW // _GCH
_GRP = 4
_SGRP = 2
_HP = 128
_NPAD = 10240
_RPS = _NPAD // _NS

_BE = 2560
_BN = _BE // _DEG
_EG = _E // _BE
_BNN = 2000
_NG = _N // _BNN


def _sp(x):
    return jnp.maximum(x, 0.0) + jnp.log1p(jnp.exp(-jnp.abs(x)))


def _mm(a, b):
    return lax.dot_general(a, b, (((1,), (0,)), ((), ())),
                           preferred_element_type=jnp.float32)


def _mmT(a, b):
    return lax.dot_general(a, b, (((0,), (0,)), ((), ())),
                           preferred_element_type=jnp.float32)


def _rep_mat(m, n):
    r = lax.broadcasted_iota(jnp.int32, (m, n), 0) // _DEG
    c = lax.broadcasted_iota(jnp.int32, (m, n), 1)
    return (r == c).astype(jnp.float32)


def _prep(params):
    p = params
    ef0w, ef0b = p['edge_fn'][0][0]['w'], p['edge_fn'][0][0]['b']
    ef1w, ef1b = p['edge_fn'][1][0]['w'], p['edge_fn'][1][0]['b']
    nf0w, nf0b = p['node_fn'][0][0]['w'], p['node_fn'][0][0]['b']
    nf1w, nf1b = p['node_fn'][1][0]['w'], p['node_fn'][1][0]['b']
    gf0w, gf0b = p['glob_fn'][0][0]['w'], p['glob_fn'][0][0]['b']
    gf1w, gf1b = p['glob_fn'][1][0]['w'], p['glob_fn'][1][0]['b']
    H = _H
    r1 = lambda b: b.reshape(1, -1)
    w = {
        'we1': p['edge_enc'][0]['w'], 'be1': r1(p['edge_enc'][0]['b']),
        'we2': p['edge_enc'][1]['w'], 'be2': r1(p['edge_enc'][1]['b']),
        'wn1': p['node_enc'][0]['w'], 'bn1': r1(p['node_enc'][0]['b']),
        'wn2': p['node_enc'][1]['w'], 'bn2': r1(p['node_enc'][1]['b']),
        'wg1': p['glob_enc'][0]['w'], 'bg1': r1(p['glob_enc'][0]['b']),
        'wg2': p['glob_enc'][1]['w'], 'bg2': r1(p['glob_enc'][1]['b']),
        'wc0': ef0w[0:H] + ef0w[H:2*H],
        'ws0': ef0w[2*H:3*H] + ef0w[3*H:4*H],
        'wr0': ef0w[4*H:5*H] + ef0w[5*H:6*H],
        'wg0e': ef0w[6*H:7*H] + ef0w[7*H:8*H],
        'b1_0': r1(ef0b),
        'w2_0': p['edge_fn'][0][1]['w'], 'b2_0': r1(p['edge_fn'][0][1]['b']),
        'w1a1': ef1w[0:H], 'w1b1': ef1w[H:2*H],
        'ws1a': ef1w[2*H:3*H], 'ws1b': ef1w[3*H:4*H],
        'wr1a': ef1w[4*H:5*H], 'wr1b': ef1w[5*H:6*H],
        'wg1a': ef1w[6*H:7*H], 'wg1b': ef1w[7*H:8*H],
        'b1_1': r1(ef1b),
        'w2_1': p['edge_fn'][1][1]['w'], 'b2_1': r1(p['edge_fn'][1][1]['b']),
        'wn0ab': nf0w[0:H] + nf0w[H:2*H],
        'wn0inc': nf0w[2*H:3*H], 'wn0og': nf0w[3*H:4*H],
        'wn0g': nf0w[4*H:5*H] + nf0w[5*H:6*H],
        'bnf1_0': r1(nf0b),
        'wnf2_0': p['node_fn'][0][1]['w'], 'bnf2_0': r1(p['node_fn'][0][1]['b']),
        'wn1a': nf1w[0:H], 'wn1b': nf1w[H:2*H],
        'wn1inc': nf1w[2*H:3*H], 'wn1og': nf1w[3*H:4*H],
        'wn1ga': nf1w[4*H:5*H], 'wn1gb': nf1w[5*H:6*H],
        'bnf1_1': r1(nf1b),
        'wnf2_1': p['node_fn'][1][1]['w'], 'bnf2_1': r1(p['node_fn'][1][1]['b']),
        'wg0n': gf0w[0:H], 'wg0e_': gf0w[H:2*H],
        'wg0g': gf0w[2*H:3*H] + gf0w[3*H:4*H],
        'bgf1_0': r1(gf0b),
        'wgf2_0': p['glob_fn'][0][1]['w'], 'bgf2_0': r1(p['glob_fn'][0][1]['b']),
        'wg1n': gf1w[0:H], 'wg1e_': gf1w[H:2*H],
        'wg1ga': gf1w[2*H:3*H], 'wg1gb': gf1w[3*H:4*H],
        'bgf1_1': r1(gf1b),
        'wgf2_1': p['glob_fn'][1][1]['w'], 'bgf2_1': r1(p['glob_fn'][1][1]['b']),
        'wd0': p['decoder'][0]['w'], 'bd0': r1(p['decoder'][0]['b']),
        'wd1': p['decoder'][1]['w'], 'bd1': r1(p['decoder'][1]['b']),
        'wd2': p['decoder'][2]['w'], 'bd2': r1(p['decoder'][2]['b']),
    }
    return w


def _whole(x):
    return pl.BlockSpec(x.shape, lambda *_: (0,) * x.ndim)


def _sc_gather(table, idx3):
    mesh = plsc.VectorSubcoreMesh(core_axis_name="c", subcore_axis_name="s")

    @functools.partial(
        pl.kernel,
        out_type=jax.ShapeDtypeStruct((_E, _HP), jnp.float32),
        mesh=mesh,
        scratch_types=[
            pltpu.VMEM((_GIT, _GCH), jnp.int32),
            pltpu.VMEM((_GRP * _GCH, _HP), jnp.float32),
            pltpu.SemaphoreType.DMA,
        ],
    )
    def k(table_hbm, idx_hbm, out_hbm, idx_v, rows_v, gsem):
        c = lax.axis_index("c")
        s = lax.axis_index("s")
        wid = s * _NC + c
        base = wid * _EPW
        pltpu.sync_copy(idx_hbm.at[wid], idx_v)

        def sup(g, carry):
            ds = [
                pltpu.async_copy(table_hbm.at[idx_v.at[g * _GRP + b]],
                                 rows_v.at[pl.ds(b * _GCH, _GCH)], gsem)
                for b in range(_GRP)
            ]
            for d in ds:
                d.wait()
            pltpu.sync_copy(
                rows_v,
                out_hbm.at[pl.ds(base + g * (_GRP * _GCH), _GRP * _GCH)])
            return carry

        lax.fori_loop(0, _GIT // _GRP, sup, 0)

    return k(table, idx3)


def _sc_scatter(ne, idx3, zer):
    mesh = plsc.VectorSubcoreMesh(core_axis_name="c", subcore_axis_name="s")

    @functools.partial(
        pl.kernel,
        out_type=jax.ShapeDtypeStruct((_NC, _NPAD, _HP), jnp.float32),
        mesh=mesh,
        scratch_types=[
            pltpu.VMEM_SHARED((_NPAD, _HP), jnp.float32),
            pltpu.VMEM((_GIT, _GCH), jnp.int32),
            pltpu.VMEM((_SGRP * _GCH, _HP), jnp.float32),
        ],
    )
    def k(ne_hbm, idx_hbm, zer_hbm, out_hbm, acc_sh, idx_v, rows_v):
        c = lax.axis_index("c")
        s = lax.axis_index("s")
        wid = s * _NC + c
        base = wid * _EPW
        pltpu.sync_copy(zer_hbm, acc_sh.at[pl.ds(s * _RPS, _RPS)])
        pltpu.sync_copy(idx_hbm.at[wid], idx_v)
        plsc.subcore_barrier()

        def sup(g, carry):
            pltpu.sync_copy(
                ne_hbm.at[pl.ds(base + g * (_SGRP * _GCH), _SGRP * _GCH)],
                rows_v)
            for b in range(_SGRP):
                pltpu.sync_copy(rows_v.at[pl.ds(b * _GCH, _GCH)],
                                acc_sh.at[idx_v.at[g * _SGRP + b]], add=True)
            return carry

        lax.fori_loop(0, _GIT // _SGRP, sup, 0)
        plsc.subcore_barrier()
        pltpu.sync_copy(acc_sh.at[pl.ds(s * _RPS, _RPS)],
                        out_hbm.at[c, pl.ds(s * _RPS, _RPS)])

    return k(ne, idx3, zer)


def _k_glob_enc(globs2, w):
    def body(g_ref, wg1, bg1, wg2, bg2, wg0e, b1_0, wn0g,
             bnf1_0, encg_ref, ge0_ref, bng0_ref):
        encg = _sp(_mm(_sp(_mm(g_ref[...], wg1[...]) + bg1[...]),
                       wg2[...]) + bg2[...])
        encg_ref[...] = encg
        ge0_ref[...] = _mm(encg, wg0e[...]) + b1_0[...]
        bng0_ref[...] = _mm(encg, wn0g[...]) + bnf1_0[...]

    shp = jax.ShapeDtypeStruct((1, _H), jnp.float32)
    return pl.pallas_call(
        body,
        out_shape=[shp, shp, shp],
    )(globs2, w['wg1'], w['bg1'], w['wg2'], w['bg2'], w['wg0e'],
      w['b1_0'], w['wn0g'], w['bnf1_0'])


def _k_node_enc(nodes, w):
    def body(x_ref, wn1, bn1, wn2, bn2, ws0, wr0, enc_ref, s0_ref, r0_ref):
        h = _sp(_mm(x_ref[...], wn1[...]) + bn1[...])
        enc = _sp(_mm(h, wn2[...]) + bn2[...])
        enc_ref[...] = enc
        s0_ref[...] = _mm(enc, ws0[...])
        r0_ref[...] = jnp.concatenate(
            [_mm(enc, wr0[...]), jnp.zeros((_BNN, _HP - _H), jnp.float32)],
            axis=1)

    outs = [jax.ShapeDtypeStruct((_N, _H), jnp.float32),
            jax.ShapeDtypeStruct((_N, _H), jnp.float32),
            jax.ShapeDtypeStruct((_N, _HP), jnp.float32)]
    ws = [w['wn1'], w['bn1'], w['wn2'], w['bn2'], w['ws0'], w['wr0']]
    return pl.pallas_call(
        body,
        grid=(_NG,),
        in_specs=[pl.BlockSpec((_BNN, _DN), lambda i: (i, 0))] +
                 [_whole(x) for x in ws],
        out_specs=[pl.BlockSpec((_BNN, _H), lambda i: (i, 0)),
                   pl.BlockSpec((_BNN, _H), lambda i: (i, 0)),
                   pl.BlockSpec((_BNN, _HP), lambda i: (i, 0))],
        out_shape=outs,
    )(nodes, *ws)


def _k_edge0(edges_f, rg0, s0, ge0, w):
    def body(xe_ref, rg_ref, s_ref, g_ref, we1, be1, we2, be2, wc0,
             w2_0, b2_0, w1a1, w1b1, ne_ref, t1_ref, og_ref):
        h = _sp(_mm(xe_ref[...], we1[...]) + be1[...])
        enc = _sp(_mm(h, we2[...]) + be2[...])
        emat = _rep_mat(_BE, _BN)
        pre = (_mm(enc, wc0[...]) + rg_ref[:, :_H] + _mm(emat, s_ref[...])
               + g_ref[...])
        ne = _sp(_mm(_sp(pre), w2_0[...]) + b2_0[...])
        ne_ref[...] = jnp.concatenate(
            [ne, jnp.zeros((_BE, _HP - _H), jnp.float32)], axis=1)
        t1_ref[...] = _mm(ne, w1a1[...]) + _mm(enc, w1b1[...])
        og_ref[...] = _mmT(emat, ne)

    ws = [w['we1'], w['be1'], w['we2'], w['be2'], w['wc0'],
          w['w2_0'], w['b2_0'], w['w1a1'], w['w1b1']]
    return pl.pallas_call(
        body,
        grid=(_EG,),
        in_specs=[pl.BlockSpec((_BE, _DE), lambda i: (i, 0)),
                  pl.BlockSpec((_BE, _HP), lambda i: (i, 0)),
                  pl.BlockSpec((_BN, _H), lambda i: (i, 0)),
                  pl.BlockSpec((1, _H), lambda i: (0, 0))] +
                 [_whole(x) for x in ws],
        out_specs=[pl.BlockSpec((_BE, _HP), lambda i: (i, 0)),
                   pl.BlockSpec((_BE, _H), lambda i: (i, 0)),
                   pl.BlockSpec((_BN, _H), lambda i: (i, 0))],
        out_shape=[jax.ShapeDtypeStruct((_E, _HP), jnp.float32),
                   jax.ShapeDtypeStruct((_E, _H), jnp.float32),
                   jax.ShapeDtypeStruct((_N, _H), jnp.float32)],
    )(edges_f, rg0, s0, ge0, *ws)


def _k_node0(encn, inc0, og0, bng0, w):
    def body(enc_ref, inc_ref, og_ref, bng_ref, wn0ab, wn0inc, wn0og,
             wnf2_0, bnf2_0, ws1a, ws1b, wr1a, wr1b,
             nn_ref, s1_ref, r1_ref, nsum_ref, esum_ref):
        enc = enc_ref[...]
        inc = (inc_ref[0] + inc_ref[1])[:, :_H]
        og = og_ref[...]
        pre = (_mm(enc, wn0ab[...]) + _mm(inc, wn0inc[...])
               + _mm(og, wn0og[...]) + bng_ref[...])
        nn = _sp(_mm(_sp(pre), wnf2_0[...]) + bnf2_0[...])
        nn_ref[...] = nn
        s1_ref[...] = _mm(nn, ws1a[...]) + _mm(enc, ws1b[...])
        r1_ref[...] = jnp.concatenate(
            [_mm(nn, wr1a[...]) + _mm(enc, wr1b[...]),
             jnp.zeros((_BNN, _HP - _H), jnp.float32)], axis=1)

        @pl.when(pl.program_id(0) == 0)
        def _():
            nsum_ref[...] = jnp.zeros_like(nsum_ref)
            esum_ref[...] = jnp.zeros_like(esum_ref)

        nsum_ref[...] += jnp.sum(nn, axis=0, keepdims=True)
        esum_ref[...] += jnp.sum(og, axis=0, keepdims=True)

    ws = [w['wn0ab'], w['wn0inc'], w['wn0og'], w['wnf2_0'], w['bnf2_0'],
          w['ws1a'], w['ws1b'], w['wr1a'], w['wr1b']]
    big = jax.ShapeDtypeStruct((_N, _H), jnp.float32)
    vec = jax.ShapeDtypeStruct((1, _H), jnp.float32)
    return pl.pallas_call(
        body,
        grid=(_NG,),
        in_specs=[pl.BlockSpec((_BNN, _H), lambda i: (i, 0)),
                  pl.BlockSpec((_NC, _BNN, _HP), lambda i: (0, i, 0)),
                  pl.BlockSpec((_BNN, _H), lambda i: (i, 0)),
                  pl.BlockSpec((1, _H), lambda i: (0, 0))] +
                 [_whole(x) for x in ws],
        out_specs=[pl.BlockSpec((_BNN, _H), lambda i: (i, 0)),
                   pl.BlockSpec((_BNN, _H), lambda i: (i, 0)),
                   pl.BlockSpec((_BNN, _HP), lambda i: (i, 0)),
                   pl.BlockSpec((1, _H), lambda i: (0, 0)),
                   pl.BlockSpec((1, _H), lambda i: (0, 0))],
        out_shape=[big, big,
                   jax.ShapeDtypeStruct((_N, _HP), jnp.float32),
                   vec, vec],
    )(encn, inc0, og0, bng0, *ws)


def _k_glob0(nsum1, esum0, encg, w):
    def body(ns_ref, es_ref, encg_ref, wg0n, wg0e_, wg0g, bgf1_0,
             wgf2_0, bgf2_0, wg1a, wg1b, b1_1, wn1ga, wn1gb, bnf1_1,
             ng0_ref, ge1_ref, bng1_ref):
        encg = encg_ref[...]
        pre = (_mm(ns_ref[...], wg0n[...]) + _mm(es_ref[...], wg0e_[...])
               + _mm(encg, wg0g[...]) + bgf1_0[...])
        ng0 = _sp(_mm(_sp(pre), wgf2_0[...]) + bgf2_0[...])
        ng0_ref[...] = ng0
        ge1_ref[...] = _mm(ng0, wg1a[...]) + _mm(encg, wg1b[...]) + b1_1[...]
        bng1_ref[...] = (_mm(ng0, wn1ga[...]) + _mm(encg, wn1gb[...])
                         + bnf1_1[...])

    shp = jax.ShapeDtypeStruct((1, _H), jnp.float32)
    return pl.pallas_call(
        body,
        out_shape=[shp, shp, shp],
    )(nsum1, esum0, encg, w['wg0n'], w['wg0e_'], w['wg0g'], w['bgf1_0'],
      w['wgf2_0'], w['bgf2_0'], w['wg1a'], w['wg1b'], w['b1_1'],
      w['wn1ga'], w['wn1gb'], w['bnf1_1'])


def _k_edge1(t1, rg1, s1, ge1, w):
    def body(t1_ref, rg_ref, s_ref, g_ref, w2_1, b2_1, ne_ref, og_ref):
        emat = _rep_mat(_BE, _BN)
        pre = (t1_ref[...] + rg_ref[:, :_H] + _mm(emat, s_ref[...])
               + g_ref[...])
        ne = _sp(_mm(_sp(pre), w2_1[...]) + b2_1[...])
        ne_ref[...] = jnp.concatenate(
            [ne, jnp.zeros((_BE, _HP - _H), jnp.float32)], axis=1)
        og_ref[...] = _mmT(emat, ne)

    ws = [w['w2_1'], w['b2_1']]
    return pl.pallas_call(
        body,
        grid=(_EG,),
        in_specs=[pl.BlockSpec((_BE, _H), lambda i: (i, 0)),
                  pl.BlockSpec((_BE, _HP), lambda i: (i, 0)),
                  pl.BlockSpec((_BN, _H), lambda i: (i, 0)),
                  pl.BlockSpec((1, _H), lambda i: (0, 0))] +
                 [_whole(x) for x in ws],
        out_specs=[pl.BlockSpec((_BE, _HP), lambda i: (i, 0)),
                   pl.BlockSpec((_BN, _H), lambda i: (i, 0))],
        out_shape=[jax.ShapeDtypeStruct((_E, _HP), jnp.float32),
                   jax.ShapeDtypeStruct((_N, _H), jnp.float32)],
    )(t1, rg1, s1, ge1, *ws)


def _k_node1(nn1, encn, inc1, og1, bng1, w):
    def body(nn_ref, enc_ref, inc_ref, og_ref, bng_ref, wn1a, wn1b,
             wn1inc, wn1og, wnf2_1, bnf2_1, nsum_ref, esum_ref):
        inc = (inc_ref[0] + inc_ref[1])[:, :_H]
        og = og_ref[...]
        pre = (_mm(nn_ref[...], wn1a[...]) + _mm(enc_ref[...], wn1b[...])
               + _mm(inc, wn1inc[...]) + _mm(og, wn1og[...]) + bng_ref[...])
        nn2 = _sp(_mm(_sp(pre), wnf2_1[...]) + bnf2_1[...])

        @pl.when(pl.program_id(0) == 0)
        def _():
            nsum_ref[...] = jnp.zeros_like(nsum_ref)
            esum_ref[...] = jnp.zeros_like(esum_ref)

        nsum_ref[...] += jnp.sum(nn2, axis=0, keepdims=True)
        esum_ref[...] += jnp.sum(og, axis=0, keepdims=True)

    ws = [w['wn1a'], w['wn1b'], w['wn1inc'], w['wn1og'], w['wnf2_1'],
          w['bnf2_1']]
    vec = jax.ShapeDtypeStruct((1, _H), jnp.float32)
    return pl.pallas_call(
        body,
        grid=(_NG,),
        in_specs=[pl.BlockSpec((_BNN, _H), lambda i: (i, 0)),
                  pl.BlockSpec((_BNN, _H), lambda i: (i, 0)),
                  pl.BlockSpec((_NC, _BNN, _HP), lambda i: (0, i, 0)),
                  pl.BlockSpec((_BNN, _H), lambda i: (i, 0)),
                  pl.BlockSpec((1, _H), lambda i: (0, 0))] +
                 [_whole(x) for x in ws],
        out_specs=[pl.BlockSpec((1, _H), lambda i: (0, 0)),
                   pl.BlockSpec((1, _H), lambda i: (0, 0))],
        out_shape=[vec, vec],
    )(nn1, encn, inc1, og1, bng1, *ws)


def _k_glob1_dec(nsum2, esum1, ng0, encg, w):
    def body(ns_ref, es_ref, ng0_ref, encg_ref, wg1n, wg1e_, wg1ga,
             wg1gb, bgf1_1, wgf2_1, bgf2_1, wd0, bd0, wd1, bd1, wd2, bd2,
             out_ref):
        pre = (_mm(ns_ref[...], wg1n[...]) + _mm(es_ref[...], wg1e_[...])
               + _mm(ng0_ref[...], wg1ga[...])
               + _mm(encg_ref[...], wg1gb[...]) + bgf1_1[...])
        ng1 = _sp(_mm(_sp(pre), wgf2_1[...]) + bgf2_1[...])
        d = _sp(_mm(ng1, wd0[...]) + bd0[...])
        d = _sp(_mm(d, wd1[...]) + bd1[...])
        out_ref[...] = _mm(d, wd2[...]) + bd2[...]

    return pl.pallas_call(
        body,
        out_shape=jax.ShapeDtypeStruct((1, 1), jnp.float32),
    )(nsum2, esum1, ng0, encg, w['wg1n'], w['wg1e_'], w['wg1ga'],
      w['wg1gb'], w['bgf1_1'], w['wgf2_1'], w['bgf2_1'],
      w['wd0'], w['bd0'], w['wd1'], w['bd1'], w['wd2'], w['bd2'])


_DIAG_GATHER_XLA = False
_DIAG_SCATTER_XLA = False


def kernel(nodes, edges, globs, edge_idx, params):
    w = _prep(params)
    edges_f = edges.reshape(_E, _DE)
    idxf = edge_idx.reshape(_E).astype(jnp.int32)
    idx3 = edge_idx.reshape(_E).astype(jnp.int32).reshape(_NW, _GIT, _GCH)

    global _sc_gather, _sc_scatter
    if _DIAG_GATHER_XLA:
        _sc_gather = lambda table, _i: jnp.take(table, idxf, axis=0)
    if _DIAG_SCATTER_XLA:
        def _sc_scatter(ne, _i, _z):
            full = jax.ops.segment_sum(ne, idxf, num_segments=_NPAD)
            return jnp.stack([full, jnp.zeros_like(full)])
    globs2 = globs.reshape(1, _DG)
    zer = jnp.zeros((_RPS, _HP), jnp.float32)

    encg, ge0, bng0 = _k_glob_enc(globs2, w)
    encn, s0, r0 = _k_node_enc(nodes, w)
    rg0 = _sc_gather(r0, idx3)
    ne0, t1, og0 = _k_edge0(edges_f, rg0, s0, ge0, w)
    inc0 = _sc_scatter(ne0, idx3, zer)
    nn1, s1, r1, nsum1, esum0 = _k_node0(encn, inc0, og0, bng0, w)
    ng0, ge1, bng1 = _k_glob0(nsum1, esum0, encg, w)
    rg1 = _sc_gather(r1, idx3)
    ne1, og1 = _k_edge1(t1, rg1, s1, ge1, w)
    inc1 = _sc_scatter(ne1, idx3, zer)
    nsum2, esum1 = _k_node1(nn1, encn, inc1, og1, bng1, w)
    out = _k_glob1_dec(nsum2, esum1, ng0, encg, w)
    return jnp.reshape(out, ())


# speedup vs baseline: 3.6624x; 1.0175x over previous
import functools

import jax
import jax.numpy as jnp
from jax import lax
from jax.experimental import pallas as pl
from jax.experimental.pallas import tpu as pltpu
from jax.experimental.pallas import tpu_sc as plsc

_N = 10000
_DEG = 32
_E = _N * _DEG
_DN = 128
_DE = 16
_DG = 16
_H = 64

_NC = 2
_NS = 16
_NW = _NC * _NS
_EPW = _E // _NW
_GCH = 100
_GIT = _EPW // _GCH
_GRP = 8
_SGRP = 2
_HP = 128
_NPAD = 10240
_RPS = _NPAD // _NS

_BE = 2560
_BN = _BE // _DEG
_EG = _E // _BE
_BNN = 2000
_NG = _N // _BNN


def _sp(x):
    return jnp.maximum(x, 0.0) + jnp.log1p(jnp.exp(-jnp.abs(x)))


def _mm(a, b):
    return lax.dot_general(a, b, (((1,), (0,)), ((), ())),
                           preferred_element_type=jnp.float32)


def _mmT(a, b):
    return lax.dot_general(a, b, (((0,), (0,)), ((), ())),
                           preferred_element_type=jnp.float32)


def _rep_mat(m, n):
    r = lax.broadcasted_iota(jnp.int32, (m, n), 0) // _DEG
    c = lax.broadcasted_iota(jnp.int32, (m, n), 1)
    return (r == c).astype(jnp.float32)


def _prep(params):
    p = params
    ef0w, ef0b = p['edge_fn'][0][0]['w'], p['edge_fn'][0][0]['b']
    ef1w, ef1b = p['edge_fn'][1][0]['w'], p['edge_fn'][1][0]['b']
    nf0w, nf0b = p['node_fn'][0][0]['w'], p['node_fn'][0][0]['b']
    nf1w, nf1b = p['node_fn'][1][0]['w'], p['node_fn'][1][0]['b']
    gf0w, gf0b = p['glob_fn'][0][0]['w'], p['glob_fn'][0][0]['b']
    gf1w, gf1b = p['glob_fn'][1][0]['w'], p['glob_fn'][1][0]['b']
    H = _H
    r1 = lambda b: b.reshape(1, -1)
    w = {
        'we1': p['edge_enc'][0]['w'], 'be1': r1(p['edge_enc'][0]['b']),
        'we2': p['edge_enc'][1]['w'], 'be2': r1(p['edge_enc'][1]['b']),
        'wn1': p['node_enc'][0]['w'], 'bn1': r1(p['node_enc'][0]['b']),
        'wn2': p['node_enc'][1]['w'], 'bn2': r1(p['node_enc'][1]['b']),
        'wg1': p['glob_enc'][0]['w'], 'bg1': r1(p['glob_enc'][0]['b']),
        'wg2': p['glob_enc'][1]['w'], 'bg2': r1(p['glob_enc'][1]['b']),
        'wc0': ef0w[0:H] + ef0w[H:2*H],
        'ws0': ef0w[2*H:3*H] + ef0w[3*H:4*H],
        'wr0': ef0w[4*H:5*H] + ef0w[5*H:6*H],
        'wg0e': ef0w[6*H:7*H] + ef0w[7*H:8*H],
        'b1_0': r1(ef0b),
        'w2_0': p['edge_fn'][0][1]['w'], 'b2_0': r1(p['edge_fn'][0][1]['b']),
        'w1a1': ef1w[0:H], 'w1b1': ef1w[H:2*H],
        'ws1a': ef1w[2*H:3*H], 'ws1b': ef1w[3*H:4*H],
        'wr1a': ef1w[4*H:5*H], 'wr1b': ef1w[5*H:6*H],
        'wg1a': ef1w[6*H:7*H], 'wg1b': ef1w[7*H:8*H],
        'b1_1': r1(ef1b),
        'w2_1': p['edge_fn'][1][1]['w'], 'b2_1': r1(p['edge_fn'][1][1]['b']),
        'wn0ab': nf0w[0:H] + nf0w[H:2*H],
        'wn0inc': nf0w[2*H:3*H], 'wn0og': nf0w[3*H:4*H],
        'wn0g': nf0w[4*H:5*H] + nf0w[5*H:6*H],
        'bnf1_0': r1(nf0b),
        'wnf2_0': p['node_fn'][0][1]['w'], 'bnf2_0': r1(p['node_fn'][0][1]['b']),
        'wn1a': nf1w[0:H], 'wn1b': nf1w[H:2*H],
        'wn1inc': nf1w[2*H:3*H], 'wn1og': nf1w[3*H:4*H],
        'wn1ga': nf1w[4*H:5*H], 'wn1gb': nf1w[5*H:6*H],
        'bnf1_1': r1(nf1b),
        'wnf2_1': p['node_fn'][1][1]['w'], 'bnf2_1': r1(p['node_fn'][1][1]['b']),
        'wg0n': gf0w[0:H], 'wg0e_': gf0w[H:2*H],
        'wg0g': gf0w[2*H:3*H] + gf0w[3*H:4*H],
        'bgf1_0': r1(gf0b),
        'wgf2_0': p['glob_fn'][0][1]['w'], 'bgf2_0': r1(p['glob_fn'][0][1]['b']),
        'wg1n': gf1w[0:H], 'wg1e_': gf1w[H:2*H],
        'wg1ga': gf1w[2*H:3*H], 'wg1gb': gf1w[3*H:4*H],
        'bgf1_1': r1(gf1b),
        'wgf2_1': p['glob_fn'][1][1]['w'], 'bgf2_1': r1(p['glob_fn'][1][1]['b']),
        'wd0': p['decoder'][0]['w'], 'bd0': r1(p['decoder'][0]['b']),
        'wd1': p['decoder'][1]['w'], 'bd1': r1(p['decoder'][1]['b']),
        'wd2': p['decoder'][2]['w'], 'bd2': r1(p['decoder'][2]['b']),
    }
    return w


def _whole(x):
    return pl.BlockSpec(x.shape, lambda *_: (0,) * x.ndim)


def _sc_gather(table, idx3):
    mesh = plsc.VectorSubcoreMesh(core_axis_name="c", subcore_axis_name="s")

    @functools.partial(
        pl.kernel,
        out_type=jax.ShapeDtypeStruct((_E, _HP), jnp.float32),
        mesh=mesh,
        scratch_types=[
            pltpu.VMEM((_GIT, _GCH), jnp.int32),
            pltpu.VMEM((_GRP * _GCH, _HP), jnp.float32),
            pltpu.SemaphoreType.DMA,
        ],
    )
    def k(table_hbm, idx_hbm, out_hbm, idx_v, rows_v, gsem):
        c = lax.axis_index("c")
        s = lax.axis_index("s")
        wid = s * _NC + c
        base = wid * _EPW
        pltpu.sync_copy(idx_hbm.at[wid], idx_v)

        def sup(g, carry):
            ds = [
                pltpu.async_copy(table_hbm.at[idx_v.at[g * _GRP + b]],
                                 rows_v.at[pl.ds(b * _GCH, _GCH)], gsem)
                for b in range(_GRP)
            ]
            for d in ds:
                d.wait()
            pltpu.sync_copy(
                rows_v,
                out_hbm.at[pl.ds(base + g * (_GRP * _GCH), _GRP * _GCH)])
            return carry

        lax.fori_loop(0, _GIT // _GRP, sup, 0)

    return k(table, idx3)


def _sc_scatter(ne, idx3, zer):
    mesh = plsc.VectorSubcoreMesh(core_axis_name="c", subcore_axis_name="s")

    @functools.partial(
        pl.kernel,
        out_type=[jax.ShapeDtypeStruct((_NPAD, _HP), jnp.float32),
                  jax.ShapeDtypeStruct((_NPAD, _HP), jnp.float32)],
        mesh=mesh,
        scratch_types=[
            pltpu.VMEM_SHARED((_NPAD, _HP), jnp.float32),
            pltpu.VMEM((_GIT, _GCH), jnp.int32),
            pltpu.VMEM((_SGRP * _GCH, _HP), jnp.float32),
        ],
    )
    def k(ne_hbm, idx_hbm, zer_hbm, out0_hbm, out1_hbm, acc_sh, idx_v,
          rows_v):
        c = lax.axis_index("c")
        s = lax.axis_index("s")
        wid = s * _NC + c
        base = wid * _EPW
        pltpu.sync_copy(zer_hbm, acc_sh.at[pl.ds(s * _RPS, _RPS)])
        pltpu.sync_copy(idx_hbm.at[wid], idx_v)
        plsc.subcore_barrier()

        def sup(g, carry):
            pltpu.sync_copy(
                ne_hbm.at[pl.ds(base + g * (_SGRP * _GCH), _SGRP * _GCH)],
                rows_v)
            for b in range(_SGRP):
                pltpu.sync_copy(rows_v.at[pl.ds(b * _GCH, _GCH)],
                                acc_sh.at[idx_v.at[g * _SGRP + b]], add=True)
            return carry

        lax.fori_loop(0, _GIT // _SGRP, sup, 0)
        plsc.subcore_barrier()

        @pl.when(c == 0)
        def _():
            pltpu.sync_copy(acc_sh.at[pl.ds(s * _RPS, _RPS)],
                            out0_hbm.at[pl.ds(s * _RPS, _RPS)])

        @pl.when(c == 1)
        def _():
            pltpu.sync_copy(acc_sh.at[pl.ds(s * _RPS, _RPS)],
                            out1_hbm.at[pl.ds(s * _RPS, _RPS)])

    return k(ne, idx3, zer)


def _k_glob_enc(globs2, w):
    def body(g_ref, wg1, bg1, wg2, bg2, wg0e, b1_0, wn0g,
             bnf1_0, encg_ref, ge0_ref, bng0_ref):
        encg = _sp(_mm(_sp(_mm(g_ref[...], wg1[...]) + bg1[...]),
                       wg2[...]) + bg2[...])
        encg_ref[...] = encg
        ge0_ref[...] = _mm(encg, wg0e[...]) + b1_0[...]
        bng0_ref[...] = _mm(encg, wn0g[...]) + bnf1_0[...]

    shp = jax.ShapeDtypeStruct((1, _H), jnp.float32)
    return pl.pallas_call(
        body,
        out_shape=[shp, shp, shp],
    )(globs2, w['wg1'], w['bg1'], w['wg2'], w['bg2'], w['wg0e'],
      w['b1_0'], w['wn0g'], w['bnf1_0'])


def _k_node_enc(nodes, w):
    def body(x_ref, wn1, bn1, wn2, bn2, ws0, wr0, enc_ref, s0_ref, r0_ref):
        h = _sp(_mm(x_ref[...], wn1[...]) + bn1[...])
        enc = _sp(_mm(h, wn2[...]) + bn2[...])
        enc_ref[...] = enc
        s0_ref[...] = _mm(enc, ws0[...])
        r0_ref[...] = jnp.concatenate(
            [_mm(enc, wr0[...]), jnp.zeros((_BNN, _HP - _H), jnp.float32)],
            axis=1)

    outs = [jax.ShapeDtypeStruct((_N, _H), jnp.float32),
            jax.ShapeDtypeStruct((_N, _H), jnp.float32),
            jax.ShapeDtypeStruct((_N, _HP), jnp.float32)]
    ws = [w['wn1'], w['bn1'], w['wn2'], w['bn2'], w['ws0'], w['wr0']]
    return pl.pallas_call(
        body,
        grid=(_NG,),
        in_specs=[pl.BlockSpec((_BNN, _DN), lambda i: (i, 0))] +
                 [_whole(x) for x in ws],
        out_specs=[pl.BlockSpec((_BNN, _H), lambda i: (i, 0)),
                   pl.BlockSpec((_BNN, _H), lambda i: (i, 0)),
                   pl.BlockSpec((_BNN, _HP), lambda i: (i, 0))],
        out_shape=outs,
    )(nodes, *ws)


def _k_edge0(edges_f, rg0, s0, ge0, w):
    def body(xe_ref, rg_ref, s_ref, g_ref, we1, be1, we2, be2, wc0,
             w2_0, b2_0, w1a1, w1b1, ne_ref, t1_ref, og_ref):
        h = _sp(_mm(xe_ref[...], we1[...]) + be1[...])
        enc = _sp(_mm(h, we2[...]) + be2[...])
        emat = _rep_mat(_BE, _BN)
        pre = (_mm(enc, wc0[...]) + rg_ref[:, :_H] + _mm(emat, s_ref[...])
               + g_ref[...])
        ne = _sp(_mm(_sp(pre), w2_0[...]) + b2_0[...])
        ne_ref[...] = jnp.concatenate(
            [ne, jnp.zeros((_BE, _HP - _H), jnp.float32)], axis=1)
        t1_ref[...] = _mm(ne, w1a1[...]) + _mm(enc, w1b1[...])
        og_ref[...] = _mmT(emat, ne)

    ws = [w['we1'], w['be1'], w['we2'], w['be2'], w['wc0'],
          w['w2_0'], w['b2_0'], w['w1a1'], w['w1b1']]
    return pl.pallas_call(
        body,
        grid=(_EG,),
        in_specs=[pl.BlockSpec((_BE, _DE), lambda i: (i, 0)),
                  pl.BlockSpec((_BE, _HP), lambda i: (i, 0)),
                  pl.BlockSpec((_BN, _H), lambda i: (i, 0)),
                  pl.BlockSpec((1, _H), lambda i: (0, 0))] +
                 [_whole(x) for x in ws],
        out_specs=[pl.BlockSpec((_BE, _HP), lambda i: (i, 0)),
                   pl.BlockSpec((_BE, _H), lambda i: (i, 0)),
                   pl.BlockSpec((_BN, _H), lambda i: (i, 0))],
        out_shape=[jax.ShapeDtypeStruct((_E, _HP), jnp.float32),
                   jax.ShapeDtypeStruct((_E, _H), jnp.float32),
                   jax.ShapeDtypeStruct((_N, _H), jnp.float32)],
    )(edges_f, rg0, s0, ge0, *ws)


def _k_node0(encn, inc0a, inc0b, og0, bng0, w):
    def body(enc_ref, inca_ref, incb_ref, og_ref, bng_ref, wn0ab, wn0inc,
             wn0og, wnf2_0, bnf2_0, ws1a, ws1b, wr1a, wr1b,
             nn_ref, s1_ref, r1_ref, nsum_ref, esum_ref):
        enc = enc_ref[...]
        inc = (inca_ref[...] + incb_ref[...])[:, :_H]
        og = og_ref[...]
        pre = (_mm(enc, wn0ab[...]) + _mm(inc, wn0inc[...])
               + _mm(og, wn0og[...]) + bng_ref[...])
        nn = _sp(_mm(_sp(pre), wnf2_0[...]) + bnf2_0[...])
        nn_ref[...] = nn
        s1_ref[...] = _mm(nn, ws1a[...]) + _mm(enc, ws1b[...])
        r1_ref[...] = jnp.concatenate(
            [_mm(nn, wr1a[...]) + _mm(enc, wr1b[...]),
             jnp.zeros((_BNN, _HP - _H), jnp.float32)], axis=1)

        @pl.when(pl.program_id(0) == 0)
        def _():
            nsum_ref[...] = jnp.zeros_like(nsum_ref)
            esum_ref[...] = jnp.zeros_like(esum_ref)

        nsum_ref[...] += jnp.sum(nn, axis=0, keepdims=True)
        esum_ref[...] += jnp.sum(og, axis=0, keepdims=True)

    ws = [w['wn0ab'], w['wn0inc'], w['wn0og'], w['wnf2_0'], w['bnf2_0'],
          w['ws1a'], w['ws1b'], w['wr1a'], w['wr1b']]
    big = jax.ShapeDtypeStruct((_N, _H), jnp.float32)
    vec = jax.ShapeDtypeStruct((1, _H), jnp.float32)
    return pl.pallas_call(
        body,
        grid=(_NG,),
        in_specs=[pl.BlockSpec((_BNN, _H), lambda i: (i, 0)),
                  pl.BlockSpec((_BNN, _HP), lambda i: (i, 0)),
                  pl.BlockSpec((_BNN, _HP), lambda i: (i, 0)),
                  pl.BlockSpec((_BNN, _H), lambda i: (i, 0)),
                  pl.BlockSpec((1, _H), lambda i: (0, 0))] +
                 [_whole(x) for x in ws],
        out_specs=[pl.BlockSpec((_BNN, _H), lambda i: (i, 0)),
                   pl.BlockSpec((_BNN, _H), lambda i: (i, 0)),
                   pl.BlockSpec((_BNN, _HP), lambda i: (i, 0)),
                   pl.BlockSpec((1, _H), lambda i: (0, 0)),
                   pl.BlockSpec((1, _H), lambda i: (0, 0))],
        out_shape=[big, big,
                   jax.ShapeDtypeStruct((_N, _HP), jnp.float32),
                   vec, vec],
    )(encn, inc0a, inc0b, og0, bng0, *ws)


def _k_glob0(nsum1, esum0, encg, w):
    def body(ns_ref, es_ref, encg_ref, wg0n, wg0e_, wg0g, bgf1_0,
             wgf2_0, bgf2_0, wg1a, wg1b, b1_1, wn1ga, wn1gb, bnf1_1,
             ng0_ref, ge1_ref, bng1_ref):
        encg = encg_ref[...]
        pre = (_mm(ns_ref[...], wg0n[...]) + _mm(es_ref[...], wg0e_[...])
               + _mm(encg, wg0g[...]) + bgf1_0[...])
        ng0 = _sp(_mm(_sp(pre), wgf2_0[...]) + bgf2_0[...])
        ng0_ref[...] = ng0
        ge1_ref[...] = _mm(ng0, wg1a[...]) + _mm(encg, wg1b[...]) + b1_1[...]
        bng1_ref[...] = (_mm(ng0, wn1ga[...]) + _mm(encg, wn1gb[...])
                         + bnf1_1[...])

    shp = jax.ShapeDtypeStruct((1, _H), jnp.float32)
    return pl.pallas_call(
        body,
        out_shape=[shp, shp, shp],
    )(nsum1, esum0, encg, w['wg0n'], w['wg0e_'], w['wg0g'], w['bgf1_0'],
      w['wgf2_0'], w['bgf2_0'], w['wg1a'], w['wg1b'], w['b1_1'],
      w['wn1ga'], w['wn1gb'], w['bnf1_1'])


def _k_edge1(t1, rg1, s1, ge1, w):
    def body(t1_ref, rg_ref, s_ref, g_ref, w2_1, b2_1, ne_ref, og_ref):
        emat = _rep_mat(_BE, _BN)
        pre = (t1_ref[...] + rg_ref[:, :_H] + _mm(emat, s_ref[...])
               + g_ref[...])
        ne = _sp(_mm(_sp(pre), w2_1[...]) + b2_1[...])
        ne_ref[...] = jnp.concatenate(
            [ne, jnp.zeros((_BE, _HP - _H), jnp.float32)], axis=1)
        og_ref[...] = _mmT(emat, ne)

    ws = [w['w2_1'], w['b2_1']]
    return pl.pallas_call(
        body,
        grid=(_EG,),
        in_specs=[pl.BlockSpec((_BE, _H), lambda i: (i, 0)),
                  pl.BlockSpec((_BE, _HP), lambda i: (i, 0)),
                  pl.BlockSpec((_BN, _H), lambda i: (i, 0)),
                  pl.BlockSpec((1, _H), lambda i: (0, 0))] +
                 [_whole(x) for x in ws],
        out_specs=[pl.BlockSpec((_BE, _HP), lambda i: (i, 0)),
                   pl.BlockSpec((_BN, _H), lambda i: (i, 0))],
        out_shape=[jax.ShapeDtypeStruct((_E, _HP), jnp.float32),
                   jax.ShapeDtypeStruct((_N, _H), jnp.float32)],
    )(t1, rg1, s1, ge1, *ws)


def _k_node1(nn1, encn, inc1a, inc1b, og1, bng1, w):
    def body(nn_ref, enc_ref, inca_ref, incb_ref, og_ref, bng_ref, wn1a,
             wn1b, wn1inc, wn1og, wnf2_1, bnf2_1, nsum_ref, esum_ref):
        inc = (inca_ref[...] + incb_ref[...])[:, :_H]
        og = og_ref[...]
        pre = (_mm(nn_ref[...], wn1a[...]) + _mm(enc_ref[...], wn1b[...])
               + _mm(inc, wn1inc[...]) + _mm(og, wn1og[...]) + bng_ref[...])
        nn2 = _sp(_mm(_sp(pre), wnf2_1[...]) + bnf2_1[...])

        @pl.when(pl.program_id(0) == 0)
        def _():
            nsum_ref[...] = jnp.zeros_like(nsum_ref)
            esum_ref[...] = jnp.zeros_like(esum_ref)

        nsum_ref[...] += jnp.sum(nn2, axis=0, keepdims=True)
        esum_ref[...] += jnp.sum(og, axis=0, keepdims=True)

    ws = [w['wn1a'], w['wn1b'], w['wn1inc'], w['wn1og'], w['wnf2_1'],
          w['bnf2_1']]
    vec = jax.ShapeDtypeStruct((1, _H), jnp.float32)
    return pl.pallas_call(
        body,
        grid=(_NG,),
        in_specs=[pl.BlockSpec((_BNN, _H), lambda i: (i, 0)),
                  pl.BlockSpec((_BNN, _H), lambda i: (i, 0)),
                  pl.BlockSpec((_BNN, _HP), lambda i: (i, 0)),
                  pl.BlockSpec((_BNN, _HP), lambda i: (i, 0)),
                  pl.BlockSpec((_BNN, _H), lambda i: (i, 0)),
                  pl.BlockSpec((1, _H), lambda i: (0, 0))] +
                 [_whole(x) for x in ws],
        out_specs=[pl.BlockSpec((1, _H), lambda i: (0, 0)),
                   pl.BlockSpec((1, _H), lambda i: (0, 0))],
        out_shape=[vec, vec],
    )(nn1, encn, inc1a, inc1b, og1, bng1, *ws)


def _k_glob1_dec(nsum2, esum1, ng0, encg, w):
    def body(ns_ref, es_ref, ng0_ref, encg_ref, wg1n, wg1e_, wg1ga,
             wg1gb, bgf1_1, wgf2_1, bgf2_1, wd0, bd0, wd1, bd1, wd2, bd2,
             out_ref):
        pre = (_mm(ns_ref[...], wg1n[...]) + _mm(es_ref[...], wg1e_[...])
               + _mm(ng0_ref[...], wg1ga[...])
               + _mm(encg_ref[...], wg1gb[...]) + bgf1_1[...])
        ng1 = _sp(_mm(_sp(pre), wgf2_1[...]) + bgf2_1[...])
        d = _sp(_mm(ng1, wd0[...]) + bd0[...])
        d = _sp(_mm(d, wd1[...]) + bd1[...])
        out_ref[...] = _mm(d, wd2[...]) + bd2[...]

    return pl.pallas_call(
        body,
        out_shape=jax.ShapeDtypeStruct((1, 1), jnp.float32),
    )(nsum2, esum1, ng0, encg, w['wg1n'], w['wg1e_'], w['wg1ga'],
      w['wg1gb'], w['bgf1_1'], w['wgf2_1'], w['bgf2_1'],
      w['wd0'], w['bd0'], w['wd1'], w['bd1'], w['wd2'], w['bd2'])


_DIAG_GATHER_XLA = False
_DIAG_SCATTER_XLA = False


def kernel(nodes, edges, globs, edge_idx, params):
    w = _prep(params)
    edges_f = edges.reshape(_E, _DE)
    idxf = edge_idx.reshape(_E).astype(jnp.int32)
    idx3 = edge_idx.reshape(_E).astype(jnp.int32).reshape(_NW, _GIT, _GCH)

    global _sc_gather, _sc_scatter
    if _DIAG_GATHER_XLA:
        _sc_gather = lambda table, _i: jnp.take(table, idxf, axis=0)
    if _DIAG_SCATTER_XLA:
        def _sc_scatter(ne, _i, _z):
            full = jax.ops.segment_sum(ne, idxf, num_segments=_NPAD)
            return full, jnp.zeros_like(full)
    globs2 = globs.reshape(1, _DG)
    zer = jnp.zeros((_RPS, _HP), jnp.float32)

    encg, ge0, bng0 = _k_glob_enc(globs2, w)
    encn, s0, r0 = _k_node_enc(nodes, w)
    rg0 = _sc_gather(r0, idx3)
    ne0, t1, og0 = _k_edge0(edges_f, rg0, s0, ge0, w)
    inc0a, inc0b = _sc_scatter(ne0, idx3, zer)
    nn1, s1, r1, nsum1, esum0 = _k_node0(encn, inc0a, inc0b, og0, bng0, w)
    ng0, ge1, bng1 = _k_glob0(nsum1, esum0, encg, w)
    rg1 = _sc_gather(r1, idx3)
    ne1, og1 = _k_edge1(t1, rg1, s1, ge1, w)
    inc1a, inc1b = _sc_scatter(ne1, idx3, zer)
    nsum2, esum1 = _k_node1(nn1, encn, inc1a, inc1b, og1, bng1, w)
    out = _k_glob1_dec(nsum2, esum1, ng0, encg, w)
    return jnp.reshape(out, ())
